# Initial kernel scaffold; baseline (speedup 1.0000x reference)
import jax
import jax.numpy as jnp
from jax.experimental import pallas as pl


def kernel(h, edge_index, W1, b1, W2, b2):
    raise NotImplementedError("write your pallas kernel here")


# speedup vs baseline: 5.1186x; 5.1186x over previous
import functools

import jax
import jax.numpy as jnp
from jax import lax
from jax.experimental import pallas as pl
from jax.experimental.pallas import tpu as pltpu
from jax.experimental.pallas import tpu_sc as plsc

_N = 10000
_E = 320000
_D = 128
_P1 = 144
_CH = 100
_NC = 2
_NS = 16
_NW = _NC * _NS
_CPW = _E // _CH // _NW
_RPT = _N // _NS
_BN = 400


def _make_segsum(P):
  mesh = plsc.VectorSubcoreMesh(core_axis_name="c", subcore_axis_name="s")

  @functools.partial(
      pl.kernel,
      out_type=jax.ShapeDtypeStruct((_NC, _N, P), jnp.float32),
      mesh=mesh,
      compiler_params=pltpu.CompilerParams(use_tc_tiling_on_sc=False),
      scratch_types=[
          pltpu.VMEM_SHARED((_N, P), jnp.float32),
          pltpu.VMEM((_CH, P), jnp.float32),
          pltpu.VMEM((_CH,), jnp.int32),
          pltpu.VMEM((_CH,), jnp.int32),
          pltpu.SemaphoreType.DMA,
      ],
  )
  def seg(x_hbm, srcm_hbm, dstm_hbm, out_hbm, acc, rows, idxg, idxs, sem):
    c = lax.axis_index("c")
    s = lax.axis_index("s")
    w = s * _NC + c

    z = jnp.zeros((16,), jnp.float32)

    def zero_row(i, carry):
      for j in range(P // 16):
        rows[i, pl.ds(j * 16, 16)] = z
      return carry

    lax.fori_loop(0, _CH, zero_row, 0)

    base = s * _RPT
    nfull = _RPT // _CH
    rem = _RPT % _CH
    for k in range(nfull):
      pltpu.sync_copy(rows, acc.at[pl.ds(base + k * _CH, _CH)])
    if rem:
      pltpu.sync_copy(rows.at[pl.ds(0, rem)],
                      acc.at[pl.ds(base + nfull * _CH, rem)])
    plsc.subcore_barrier()

    def body(j, carry):
      r = w * _CPW + j
      pltpu.sync_copy(srcm_hbm.at[r], idxg)
      pltpu.sync_copy(dstm_hbm.at[r], idxs)
      pltpu.async_copy(x_hbm.at[idxg], rows, sem).wait()
      pltpu.sync_copy(rows, acc.at[idxs], add=True)
      return carry

    lax.fori_loop(0, _CPW, body, 0)
    plsc.subcore_barrier()

    for k in range(nfull):
      pltpu.sync_copy(acc.at[pl.ds(base + k * _CH, _CH)], rows)
      pltpu.sync_copy(rows, out_hbm.at[c, pl.ds(base + k * _CH, _CH)])
    if rem:
      off = base + nfull * _CH
      pltpu.sync_copy(acc.at[pl.ds(off, rem)], rows.at[pl.ds(0, rem)])
      pltpu.sync_copy(rows.at[pl.ds(0, rem)], out_hbm.at[c, pl.ds(off, rem)])

  return seg


_seg1 = _make_segsum(_P1)
_seg2 = _make_segsum(_D)


def _mm_a(h, W1, b1):
  def body(h_ref, w_ref, b_ref, o_ref):
    y = jnp.dot(h_ref[...], w_ref[...],
                preferred_element_type=jnp.float32,
                precision=lax.Precision.HIGHEST) + b_ref[...]
    lane = lax.broadcasted_iota(jnp.int32, (_BN, _P1 - _D), 1)
    pad = jnp.where(lane == 0, 1.0, 0.0).astype(jnp.float32)
    o_ref[...] = jnp.concatenate([y, pad], axis=1)

  return pl.pallas_call(
      body,
      grid=(_N // _BN,),
      in_specs=[
          pl.BlockSpec((_BN, _D), lambda i: (i, 0)),
          pl.BlockSpec((_D, _D), lambda i: (0, 0)),
          pl.BlockSpec((1, _D), lambda i: (0, 0)),
      ],
      out_specs=pl.BlockSpec((_BN, _P1), lambda i: (i, 0)),
      out_shape=jax.ShapeDtypeStruct((_N, _P1), jnp.float32),
  )(h, W1, b1)


def _mm_b(acc1, W2, b2):
  def body(a_ref, w_ref, b_ref, x2_ref, deg_ref):
    a = a_ref[...]
    sfull = a[0] + a[1]
    deg = jnp.maximum(sfull[:, _D:_D + 1], 1.0)
    h1 = jnp.maximum(sfull[:, :_D] / deg, 0.0)
    x2_ref[...] = jnp.dot(h1, w_ref[...],
                          preferred_element_type=jnp.float32,
                          precision=lax.Precision.HIGHEST) + b_ref[...]
    deg_ref[...] = jnp.broadcast_to(deg, (_BN, _D))

  return pl.pallas_call(
      body,
      grid=(_N // _BN,),
      in_specs=[
          pl.BlockSpec((_NC, _BN, _P1), lambda i: (0, i, 0)),
          pl.BlockSpec((_D, _D), lambda i: (0, 0)),
          pl.BlockSpec((1, _D), lambda i: (0, 0)),
      ],
      out_specs=[
          pl.BlockSpec((_BN, _D), lambda i: (i, 0)),
          pl.BlockSpec((_BN, _D), lambda i: (i, 0)),
      ],
      out_shape=[
          jax.ShapeDtypeStruct((_N, _D), jnp.float32),
          jax.ShapeDtypeStruct((_N, _D), jnp.float32),
      ],
  )(acc1, W2, b2)


def _mm_c(acc2, degb):
  def body(a_ref, deg_ref, o_ref):
    a = a_ref[...]
    s2 = a[0] + a[1]
    h2 = jnp.maximum(s2 / deg_ref[...], 0.0)
    nrm = jnp.sqrt(jnp.sum(h2 * h2, axis=1, keepdims=True))
    o_ref[...] = h2 / jnp.maximum(nrm, 1e-12)

  return pl.pallas_call(
      body,
      grid=(_N // _BN,),
      in_specs=[
          pl.BlockSpec((_NC, _BN, _D), lambda i: (0, i, 0)),
          pl.BlockSpec((_BN, _D), lambda i: (i, 0)),
      ],
      out_specs=pl.BlockSpec((_BN, _D), lambda i: (i, 0)),
      out_shape=jax.ShapeDtypeStruct((_N, _D), jnp.float32),
  )(acc2, degb)


def kernel(h, edge_index, W1, b1, W2, b2):
  src = edge_index[0].reshape(_E // _CH, _CH)
  dst = edge_index[1].reshape(_E // _CH, _CH)
  x1 = _mm_a(h, W1, b1.reshape(1, _D))
  acc1 = _seg1(x1, src, dst)
  x2, degb = _mm_b(acc1, W2, b2.reshape(1, _D))
  acc2 = _seg2(x2, src, dst)
  return _mm_c(acc2, degb)


# speedup vs baseline: 9.6026x; 1.8760x over previous
import functools

import jax
import jax.numpy as jnp
from jax import lax
from jax.experimental import pallas as pl
from jax.experimental.pallas import tpu as pltpu
from jax.experimental.pallas import tpu_sc as plsc

_N = 10000
_E = 320000
_D = 128
_P1 = 144
_CH = 100
_NC = 2
_NS = 16
_NW = _NC * _NS
_CPW = _E // _CH // _NW
_RPT = _N // _NS
_BN = 400


def _make_segsum(P):
  mesh = plsc.VectorSubcoreMesh(core_axis_name="c", subcore_axis_name="s")

  @functools.partial(
      pl.kernel,
      out_type=jax.ShapeDtypeStruct((_NC, _N, P), jnp.float32),
      mesh=mesh,
      compiler_params=pltpu.CompilerParams(use_tc_tiling_on_sc=False),
      scratch_types=[
          pltpu.VMEM_SHARED((_N, P), jnp.float32),
          pltpu.VMEM((_CH, P), jnp.float32),
          pltpu.VMEM((_CH, P), jnp.float32),
          pltpu.VMEM((_CPW // 2, _CH), jnp.int32),
          pltpu.VMEM((_CPW // 2, _CH), jnp.int32),
          pltpu.SemaphoreType.DMA,
          pltpu.SemaphoreType.DMA,
      ],
  )
  def seg(x_hbm, srcm_hbm, dstm_hbm, out_hbm, acc, rows0, rows1,
          idxsrc, idxdst, sem0, sem1):
    c = lax.axis_index("c")
    s = lax.axis_index("s")
    w = s * _NC + c
    rbuf = (rows0, rows1)
    sems = (sem0, sem1)

    z = jnp.zeros((16,), jnp.float32)

    def zero_row(i, carry):
      for j in range(P // 16):
        rows0[i, pl.ds(j * 16, 16)] = z
      return carry

    lax.fori_loop(0, _CH, zero_row, 0)

    base = s * _RPT
    nfull = _RPT // _CH
    rem = _RPT % _CH
    for k in range(nfull):
      pltpu.sync_copy(rows0, acc.at[pl.ds(base + k * _CH, _CH)])
    if rem:
      pltpu.sync_copy(rows0.at[pl.ds(0, rem)],
                      acc.at[pl.ds(base + nfull * _CH, rem)])
    plsc.subcore_barrier()

    def gstart(j, b):
      pltpu.async_copy(x_hbm.at[idxsrc.at[j]], rbuf[b], sems[b])

    def gwait(j, b):
      pltpu.make_async_copy(x_hbm.at[idxsrc.at[j]], rbuf[b], sems[b]).wait()

    def scat(j, b):
      pltpu.sync_copy(rbuf[b], acc.at[idxdst.at[j]], add=True)

    ih = _CPW // 2
    for half in range(2):
      pltpu.sync_copy(srcm_hbm.at[pl.ds(w * _CPW + half * ih, ih)], idxsrc)
      pltpu.sync_copy(dstm_hbm.at[pl.ds(w * _CPW + half * ih, ih)], idxdst)
      gstart(0, 0)
      gstart(1, 1)

      def body(t, carry):
        j = t * 2
        gwait(j, 0)
        scat(j, 0)
        gstart(j + 2, 0)
        gwait(j + 1, 1)
        scat(j + 1, 1)
        gstart(j + 3, 1)
        return carry

      lax.fori_loop(0, ih // 2 - 1, body, 0)
      gwait(ih - 2, 0)
      scat(ih - 2, 0)
      gwait(ih - 1, 1)
      scat(ih - 1, 1)
    plsc.subcore_barrier()

    for k in range(nfull):
      pltpu.sync_copy(acc.at[pl.ds(base + k * _CH, _CH)], rows0)
      pltpu.sync_copy(rows0, out_hbm.at[c, pl.ds(base + k * _CH, _CH)])
    if rem:
      off = base + nfull * _CH
      pltpu.sync_copy(acc.at[pl.ds(off, rem)], rows0.at[pl.ds(0, rem)])
      pltpu.sync_copy(rows0.at[pl.ds(0, rem)], out_hbm.at[c, pl.ds(off, rem)])

  return seg


_seg1 = _make_segsum(_P1)
_seg2 = _make_segsum(_D)


def _mm_a(h, W1, b1):
  def body(h_ref, w_ref, b_ref, o_ref):
    y = jnp.dot(h_ref[...], w_ref[...],
                preferred_element_type=jnp.float32,
                precision=lax.Precision.HIGHEST) + b_ref[...]
    lane = lax.broadcasted_iota(jnp.int32, (_BN, _P1 - _D), 1)
    pad = jnp.where(lane == 0, 1.0, 0.0).astype(jnp.float32)
    o_ref[...] = jnp.concatenate([y, pad], axis=1)

  return pl.pallas_call(
      body,
      grid=(_N // _BN,),
      in_specs=[
          pl.BlockSpec((_BN, _D), lambda i: (i, 0)),
          pl.BlockSpec((_D, _D), lambda i: (0, 0)),
          pl.BlockSpec((1, _D), lambda i: (0, 0)),
      ],
      out_specs=pl.BlockSpec((_BN, _P1), lambda i: (i, 0)),
      out_shape=jax.ShapeDtypeStruct((_N, _P1), jnp.float32),
  )(h, W1, b1)


def _mm_b(acc1, W2, b2):
  def body(a_ref, w_ref, b_ref, x2_ref, deg_ref):
    a = a_ref[...]
    sfull = a[0] + a[1]
    deg = jnp.maximum(sfull[:, _D:_D + 1], 1.0)
    h1 = jnp.maximum(sfull[:, :_D] / deg, 0.0)
    x2_ref[...] = jnp.dot(h1, w_ref[...],
                          preferred_element_type=jnp.float32,
                          precision=lax.Precision.HIGHEST) + b_ref[...]
    deg_ref[...] = jnp.broadcast_to(deg, (_BN, _D))

  return pl.pallas_call(
      body,
      grid=(_N // _BN,),
      in_specs=[
          pl.BlockSpec((_NC, _BN, _P1), lambda i: (0, i, 0)),
          pl.BlockSpec((_D, _D), lambda i: (0, 0)),
          pl.BlockSpec((1, _D), lambda i: (0, 0)),
      ],
      out_specs=[
          pl.BlockSpec((_BN, _D), lambda i: (i, 0)),
          pl.BlockSpec((_BN, _D), lambda i: (i, 0)),
      ],
      out_shape=[
          jax.ShapeDtypeStruct((_N, _D), jnp.float32),
          jax.ShapeDtypeStruct((_N, _D), jnp.float32),
      ],
  )(acc1, W2, b2)


def _mm_c(acc2, degb):
  def body(a_ref, deg_ref, o_ref):
    a = a_ref[...]
    s2 = a[0] + a[1]
    h2 = jnp.maximum(s2 / deg_ref[...], 0.0)
    nrm = jnp.sqrt(jnp.sum(h2 * h2, axis=1, keepdims=True))
    o_ref[...] = h2 / jnp.maximum(nrm, 1e-12)

  return pl.pallas_call(
      body,
      grid=(_N // _BN,),
      in_specs=[
          pl.BlockSpec((_NC, _BN, _D), lambda i: (0, i, 0)),
          pl.BlockSpec((_BN, _D), lambda i: (i, 0)),
      ],
      out_specs=pl.BlockSpec((_BN, _D), lambda i: (i, 0)),
      out_shape=jax.ShapeDtypeStruct((_N, _D), jnp.float32),
  )(acc2, degb)


def kernel(h, edge_index, W1, b1, W2, b2):
  src = edge_index[0].reshape(_E // _CH, _CH)
  dst = edge_index[1].reshape(_E // _CH, _CH)
  x1 = _mm_a(h, W1, b1.reshape(1, _D))
  acc1 = _seg1(x1, src, dst)
  x2, degb = _mm_b(acc1, W2, b2.reshape(1, _D))
  acc2 = _seg2(x2, src, dst)
  return _mm_c(acc2, degb)


# speedup vs baseline: 10.6318x; 1.1072x over previous
import functools

import jax
import jax.numpy as jnp
from jax import lax
from jax.experimental import pallas as pl
from jax.experimental.pallas import tpu as pltpu
from jax.experimental.pallas import tpu_sc as plsc

_N = 10000
_E = 320000
_D = 128
_P1 = 144
_CH = 100
_NC = 2
_NS = 16
_NW = _NC * _NS
_CPW = _E // _CH // _NW
_RPT = _N // _NS
_BN = 1000


def _make_segsum(P):
  mesh = plsc.VectorSubcoreMesh(core_axis_name="c", subcore_axis_name="s")

  @functools.partial(
      pl.kernel,
      out_type=jax.ShapeDtypeStruct((_NC, _N, P), jnp.float32),
      mesh=mesh,
      compiler_params=pltpu.CompilerParams(use_tc_tiling_on_sc=False),
      scratch_types=[
          pltpu.VMEM_SHARED((_N, P), jnp.float32),
          pltpu.VMEM((_CH, P), jnp.float32),
          pltpu.VMEM((_CH, P), jnp.float32),
          pltpu.VMEM((_CPW // 2, _CH), jnp.int32),
          pltpu.VMEM((_CPW // 2, _CH), jnp.int32),
          pltpu.SemaphoreType.DMA,
          pltpu.SemaphoreType.DMA,
      ],
  )
  def seg(x_hbm, edge_hbm, out_hbm, acc, rows0, rows1,
          idxsrc, idxdst, sem0, sem1):
    c = lax.axis_index("c")
    s = lax.axis_index("s")
    w = s * _NC + c
    rbuf = (rows0, rows1)
    sems = (sem0, sem1)

    z = jnp.zeros((16,), jnp.float32)

    def zero_row(i, carry):
      for j in range(P // 16):
        rows0[i, pl.ds(j * 16, 16)] = z
      return carry

    lax.fori_loop(0, _CH, zero_row, 0)

    base = s * _RPT
    nfull = _RPT // _CH
    rem = _RPT % _CH
    for k in range(nfull):
      pltpu.sync_copy(rows0, acc.at[pl.ds(base + k * _CH, _CH)])
    if rem:
      pltpu.sync_copy(rows0.at[pl.ds(0, rem)],
                      acc.at[pl.ds(base + nfull * _CH, rem)])
    plsc.subcore_barrier()

    def gstart(j, b):
      pltpu.async_copy(x_hbm.at[idxsrc.at[j]], rbuf[b], sems[b])

    def gwait(j, b):
      pltpu.make_async_copy(x_hbm.at[idxsrc.at[j]], rbuf[b], sems[b]).wait()

    def scat(j, b):
      pltpu.sync_copy(rbuf[b], acc.at[idxdst.at[j]], add=True)

    ih = _CPW // 2
    for half in range(2):
      pltpu.sync_copy(edge_hbm.at[0, pl.ds(w * _CPW + half * ih, ih)], idxsrc)
      pltpu.sync_copy(edge_hbm.at[1, pl.ds(w * _CPW + half * ih, ih)], idxdst)
      gstart(0, 0)
      gstart(1, 1)

      def body(t, carry):
        j = t * 2
        gwait(j, 0)
        scat(j, 0)
        gstart(j + 2, 0)
        gwait(j + 1, 1)
        scat(j + 1, 1)
        gstart(j + 3, 1)
        return carry

      lax.fori_loop(0, ih // 2 - 1, body, 0)
      gwait(ih - 2, 0)
      scat(ih - 2, 0)
      gwait(ih - 1, 1)
      scat(ih - 1, 1)
    plsc.subcore_barrier()

    for k in range(nfull):
      pltpu.sync_copy(acc.at[pl.ds(base + k * _CH, _CH)], rows0)
      pltpu.sync_copy(rows0, out_hbm.at[c, pl.ds(base + k * _CH, _CH)])
    if rem:
      off = base + nfull * _CH
      pltpu.sync_copy(acc.at[pl.ds(off, rem)], rows0.at[pl.ds(0, rem)])
      pltpu.sync_copy(rows0.at[pl.ds(0, rem)], out_hbm.at[c, pl.ds(off, rem)])

  return seg


_seg1 = _make_segsum(_P1)
_seg2 = _make_segsum(_D)


def _mm_a(h, W1, b1):
  def body(h_ref, w_ref, b_ref, o_ref):
    y = jnp.dot(h_ref[...], w_ref[...],
                preferred_element_type=jnp.float32) + b_ref[...]
    lane = lax.broadcasted_iota(jnp.int32, (_BN, _P1 - _D), 1)
    pad = jnp.where(lane == 0, 1.0, 0.0).astype(jnp.float32)
    o_ref[...] = jnp.concatenate([y, pad], axis=1)

  return pl.pallas_call(
      body,
      grid=(_N // _BN,),
      in_specs=[
          pl.BlockSpec((_BN, _D), lambda i: (i, 0)),
          pl.BlockSpec((_D, _D), lambda i: (0, 0)),
          pl.BlockSpec((1, _D), lambda i: (0, 0)),
      ],
      out_specs=pl.BlockSpec((_BN, _P1), lambda i: (i, 0)),
      out_shape=jax.ShapeDtypeStruct((_N, _P1), jnp.float32),
  )(h, W1, b1)


def _mm_b(acc1, W2, b2):
  def body(a_ref, w_ref, b_ref, x2_ref, deg_ref):
    a = a_ref[...]
    sfull = a[0] + a[1]
    deg = jnp.maximum(sfull[:, _D:_D + 1], 1.0)
    h1 = jnp.maximum(sfull[:, :_D] / deg, 0.0)
    x2_ref[...] = jnp.dot(h1, w_ref[...],
                          preferred_element_type=jnp.float32) + b_ref[...]
    deg_ref[...] = jnp.broadcast_to(deg, (_BN, _D))

  return pl.pallas_call(
      body,
      grid=(_N // _BN,),
      in_specs=[
          pl.BlockSpec((_NC, _BN, _P1), lambda i: (0, i, 0)),
          pl.BlockSpec((_D, _D), lambda i: (0, 0)),
          pl.BlockSpec((1, _D), lambda i: (0, 0)),
      ],
      out_specs=[
          pl.BlockSpec((_BN, _D), lambda i: (i, 0)),
          pl.BlockSpec((_BN, _D), lambda i: (i, 0)),
      ],
      out_shape=[
          jax.ShapeDtypeStruct((_N, _D), jnp.float32),
          jax.ShapeDtypeStruct((_N, _D), jnp.float32),
      ],
  )(acc1, W2, b2)


def _mm_c(acc2, degb):
  def body(a_ref, deg_ref, o_ref):
    a = a_ref[...]
    s2 = a[0] + a[1]
    h2 = jnp.maximum(s2 / deg_ref[...], 0.0)
    nrm = jnp.sqrt(jnp.sum(h2 * h2, axis=1, keepdims=True))
    o_ref[...] = h2 / jnp.maximum(nrm, 1e-12)

  return pl.pallas_call(
      body,
      grid=(_N // _BN,),
      in_specs=[
          pl.BlockSpec((_NC, _BN, _D), lambda i: (0, i, 0)),
          pl.BlockSpec((_BN, _D), lambda i: (i, 0)),
      ],
      out_specs=pl.BlockSpec((_BN, _D), lambda i: (i, 0)),
      out_shape=jax.ShapeDtypeStruct((_N, _D), jnp.float32),
  )(acc2, degb)


def kernel(h, edge_index, W1, b1, W2, b2):
  edge3 = edge_index.reshape(2, _E // _CH, _CH)
  x1 = _mm_a(h, W1, b1.reshape(1, _D))
  acc1 = _seg1(x1, edge3)
  x2, degb = _mm_b(acc1, W2, b2.reshape(1, _D))
  acc2 = _seg2(x2, edge3)
  return _mm_c(acc2, degb)


# speedup vs baseline: 11.4397x; 1.0760x over previous
import functools

import jax
import jax.numpy as jnp
from jax import lax
from jax.experimental import pallas as pl
from jax.experimental.pallas import tpu as pltpu
from jax.experimental.pallas import tpu_sc as plsc

_N = 10000
_E = 320000
_D = 128
_P1 = 144
_CH = 100
_NC = 2
_NS = 16
_NW = _NC * _NS
_CPW = _E // _CH // _NW
_RPT = _N // _NS
_BN = 1000


def _make_segsum(with_deg):
  mesh = plsc.VectorSubcoreMesh(core_axis_name="c", subcore_axis_name="s")
  out_type = [jax.ShapeDtypeStruct((_NC, _N, _D), jnp.float32)]
  scratch = [
      pltpu.VMEM_SHARED((_N, _D), jnp.float32),
      pltpu.VMEM((_CH, _D), jnp.float32),
      pltpu.VMEM((_CH, _D), jnp.float32),
      pltpu.VMEM((_CPW // 2, _CH), jnp.int32),
      pltpu.VMEM((_CPW // 2, _CH), jnp.int32),
      pltpu.SemaphoreType.DMA,
      pltpu.SemaphoreType.DMA,
  ]
  if with_deg:
    out_type.append(jax.ShapeDtypeStruct((_NC, _N, 16), jnp.float32))
    scratch += [
        pltpu.VMEM_SHARED((_N, 16), jnp.float32),
        pltpu.VMEM((_CH, 16), jnp.float32),
        pltpu.SemaphoreType.DMA,
    ]

  @functools.partial(
      pl.kernel,
      out_type=out_type,
      mesh=mesh,
      compiler_params=pltpu.CompilerParams(use_tc_tiling_on_sc=False),
      scratch_types=scratch,
  )
  def seg(*refs):
    if with_deg:
      (x_hbm, edge_hbm, out_hbm, out16_hbm, acc, rows0, rows1,
       idxsrc, idxdst, sem0, sem1, acc16, obuf, sem2) = refs
    else:
      (x_hbm, edge_hbm, out_hbm, acc, rows0, rows1,
       idxsrc, idxdst, sem0, sem1) = refs
    c = lax.axis_index("c")
    s = lax.axis_index("s")
    w = s * _NC + c
    rbuf = (rows0, rows1)
    sems = (sem0, sem1)

    z = jnp.zeros((16,), jnp.float32)

    def zero_row(i, carry):
      for j in range(_D // 16):
        rows0[i, pl.ds(j * 16, 16)] = z
      return carry

    lax.fori_loop(0, _CH, zero_row, 0)

    base = s * _RPT
    nfull = _RPT // _CH
    rem = _RPT % _CH
    for k in range(nfull):
      pltpu.sync_copy(rows0, acc.at[pl.ds(base + k * _CH, _CH)])
    if rem:
      pltpu.sync_copy(rows0.at[pl.ds(0, rem)],
                      acc.at[pl.ds(base + nfull * _CH, rem)])
    if with_deg:
      def zero_o(i, carry):
        obuf[i, pl.ds(0, 16)] = z
        return carry

      lax.fori_loop(0, _CH, zero_o, 0)
      for k in range(nfull):
        pltpu.sync_copy(obuf, acc16.at[pl.ds(base + k * _CH, _CH)])
      if rem:
        pltpu.sync_copy(obuf.at[pl.ds(0, rem)],
                        acc16.at[pl.ds(base + nfull * _CH, rem)])
      one = jnp.ones((16,), jnp.float32)

      def ones_o(i, carry):
        obuf[i, pl.ds(0, 16)] = one
        return carry

      lax.fori_loop(0, _CH, ones_o, 0)
    plsc.subcore_barrier()

    def gstart(j, b):
      pltpu.async_copy(x_hbm.at[idxsrc.at[j]], rbuf[b], sems[b])

    def gwait(j, b):
      pltpu.make_async_copy(x_hbm.at[idxsrc.at[j]], rbuf[b], sems[b]).wait()

    def scat(j, b):
      pltpu.sync_copy(rbuf[b], acc.at[idxdst.at[j]], add=True)

    ih = _CPW // 2
    for half in range(2):
      pltpu.sync_copy(edge_hbm.at[0, pl.ds(w * _CPW + half * ih, ih)], idxsrc)
      pltpu.sync_copy(edge_hbm.at[1, pl.ds(w * _CPW + half * ih, ih)], idxdst)
      gstart(0, 0)
      gstart(1, 1)

      def body(t, carry):
        j = t * 2
        gwait(j, 0)
        scat(j, 0)
        gstart(j + 2, 0)
        gwait(j + 1, 1)
        scat(j + 1, 1)
        gstart(j + 3, 1)
        return carry

      lax.fori_loop(0, ih // 2 - 1, body, 0)
      gwait(ih - 2, 0)
      scat(ih - 2, 0)
      gwait(ih - 1, 1)
      scat(ih - 1, 1)

    if with_deg:
      for half in range(2):
        pltpu.sync_copy(edge_hbm.at[1, pl.ds(w * _CPW + half * ih, ih)],
                        idxdst)

        def dfire(j, carry):
          pltpu.async_copy(obuf, acc16.at[idxdst.at[j]], sem2, add=True)
          return carry

        lax.fori_loop(0, ih, dfire, 0)

        def ddrain(j, carry):
          pltpu.make_async_copy(obuf, acc16.at[idxdst.at[0]], sem2).wait()
          return carry

        lax.fori_loop(0, ih, ddrain, 0)
    plsc.subcore_barrier()

    for k in range(nfull):
      pltpu.sync_copy(acc.at[pl.ds(base + k * _CH, _CH)], rows0)
      pltpu.sync_copy(rows0, out_hbm.at[c, pl.ds(base + k * _CH, _CH)])
    if rem:
      off = base + nfull * _CH
      pltpu.sync_copy(acc.at[pl.ds(off, rem)], rows0.at[pl.ds(0, rem)])
      pltpu.sync_copy(rows0.at[pl.ds(0, rem)], out_hbm.at[c, pl.ds(off, rem)])
    if with_deg:
      for k in range(nfull):
        pltpu.sync_copy(acc16.at[pl.ds(base + k * _CH, _CH)], obuf)
        pltpu.sync_copy(obuf, out16_hbm.at[c, pl.ds(base + k * _CH, _CH)])
      if rem:
        off = base + nfull * _CH
        pltpu.sync_copy(acc16.at[pl.ds(off, rem)], obuf.at[pl.ds(0, rem)])
        pltpu.sync_copy(obuf.at[pl.ds(0, rem)],
                        out16_hbm.at[c, pl.ds(off, rem)])

  return seg


_seg1 = _make_segsum(True)
_seg2 = _make_segsum(False)


def _mm_a(h, W1, b1):
  def body(h_ref, w_ref, b_ref, o_ref):
    o_ref[...] = jnp.dot(h_ref[...], w_ref[...],
                         preferred_element_type=jnp.float32) + b_ref[...]

  return pl.pallas_call(
      body,
      grid=(_N // _BN,),
      in_specs=[
          pl.BlockSpec((_BN, _D), lambda i: (i, 0)),
          pl.BlockSpec((_D, _D), lambda i: (0, 0)),
          pl.BlockSpec((1, _D), lambda i: (0, 0)),
      ],
      out_specs=pl.BlockSpec((_BN, _D), lambda i: (i, 0)),
      out_shape=jax.ShapeDtypeStruct((_N, _D), jnp.float32),
  )(h, W1, b1)


def _mm_b(acc1, deg16, W2, b2):
  def body(a_ref, d_ref, w_ref, b_ref, x2_ref, deg_ref):
    a = a_ref[...]
    d = d_ref[...]
    deg = jnp.maximum(d[0, :, 0:1] + d[1, :, 0:1], 1.0)
    h1 = jnp.maximum((a[0] + a[1]) / deg, 0.0)
    x2_ref[...] = jnp.dot(h1, w_ref[...],
                          preferred_element_type=jnp.float32) + b_ref[...]
    deg_ref[...] = jnp.broadcast_to(deg, (_BN, _D))

  return pl.pallas_call(
      body,
      grid=(_N // _BN,),
      in_specs=[
          pl.BlockSpec((_NC, _BN, _D), lambda i: (0, i, 0)),
          pl.BlockSpec((_NC, _BN, 16), lambda i: (0, i, 0)),
          pl.BlockSpec((_D, _D), lambda i: (0, 0)),
          pl.BlockSpec((1, _D), lambda i: (0, 0)),
      ],
      out_specs=[
          pl.BlockSpec((_BN, _D), lambda i: (i, 0)),
          pl.BlockSpec((_BN, _D), lambda i: (i, 0)),
      ],
      out_shape=[
          jax.ShapeDtypeStruct((_N, _D), jnp.float32),
          jax.ShapeDtypeStruct((_N, _D), jnp.float32),
      ],
  )(acc1, deg16, W2, b2)


def _mm_c(acc2, degb):
  def body(a_ref, deg_ref, o_ref):
    a = a_ref[...]
    s2 = a[0] + a[1]
    h2 = jnp.maximum(s2 / deg_ref[...], 0.0)
    nrm = jnp.sqrt(jnp.sum(h2 * h2, axis=1, keepdims=True))
    o_ref[...] = h2 / jnp.maximum(nrm, 1e-12)

  return pl.pallas_call(
      body,
      grid=(_N // _BN,),
      in_specs=[
          pl.BlockSpec((_NC, _BN, _D), lambda i: (0, i, 0)),
          pl.BlockSpec((_BN, _D), lambda i: (i, 0)),
      ],
      out_specs=pl.BlockSpec((_BN, _D), lambda i: (i, 0)),
      out_shape=jax.ShapeDtypeStruct((_N, _D), jnp.float32),
  )(acc2, degb)


def kernel(h, edge_index, W1, b1, W2, b2):
  edge3 = edge_index.reshape(2, _E // _CH, _CH)
  x1 = _mm_a(h, W1, b1.reshape(1, _D))
  acc1, deg16 = _seg1(x1, edge3)
  x2, degb = _mm_b(acc1, deg16, W2, b2.reshape(1, _D))
  acc2, = _seg2(x2, edge3)
  return _mm_c(acc2, degb)


# speedup vs baseline: 11.7382x; 1.0261x over previous
import functools

import jax
import jax.numpy as jnp
from jax import lax
from jax.experimental import pallas as pl
from jax.experimental.pallas import tpu as pltpu
from jax.experimental.pallas import tpu_sc as plsc

_N = 10000
_E = 320000
_D = 128
_P1 = 144
_CH = 100
_NC = 2
_NS = 16
_NW = _NC * _NS
_CPW = _E // _CH // _NW
_RPT = _N // _NS
_BN = 1000


def _make_segsum(with_deg):
  mesh = plsc.VectorSubcoreMesh(core_axis_name="c", subcore_axis_name="s")
  out_type = [jax.ShapeDtypeStruct((_NC, _N, _D), jnp.float32)]
  scratch = [
      pltpu.VMEM_SHARED((_N, _D), jnp.float32),
      pltpu.VMEM((_CH, _D), jnp.float32),
      pltpu.VMEM((_CH, _D), jnp.float32),
      pltpu.VMEM((_CPW // 2, _CH), jnp.int32),
      pltpu.VMEM((_CPW // 2, _CH), jnp.int32),
      pltpu.SemaphoreType.DMA,
      pltpu.SemaphoreType.DMA,
  ]
  if with_deg:
    out_type.append(jax.ShapeDtypeStruct((_NC, _N, 16), jnp.float32))
    scratch += [
        pltpu.VMEM_SHARED((_N, 16), jnp.float32),
        pltpu.VMEM((_CH, 16), jnp.float32),
        pltpu.SemaphoreType.DMA,
    ]

  @functools.partial(
      pl.kernel,
      out_type=out_type,
      mesh=mesh,
      compiler_params=pltpu.CompilerParams(use_tc_tiling_on_sc=False),
      scratch_types=scratch,
  )
  def seg(*refs):
    if with_deg:
      (x_hbm, edge_hbm, out_hbm, out16_hbm, acc, rows0, rows1,
       idxsrc, idxdst, sem0, sem1, acc16, obuf, sem2) = refs
    else:
      (x_hbm, edge_hbm, out_hbm, acc, rows0, rows1,
       idxsrc, idxdst, sem0, sem1) = refs
    c = lax.axis_index("c")
    s = lax.axis_index("s")
    w = s * _NC + c
    rbuf = (rows0, rows1)
    sems = (sem0, sem1)

    z = jnp.zeros((16,), jnp.float32)

    def zero_row(i, carry):
      for j in range(_D // 16):
        rows0[i, pl.ds(j * 16, 16)] = z
      return carry

    lax.fori_loop(0, _CH, zero_row, 0)

    base = s * _RPT
    nfull = _RPT // _CH
    rem = _RPT % _CH
    for k in range(nfull):
      pltpu.sync_copy(rows0, acc.at[pl.ds(base + k * _CH, _CH)])
    if rem:
      pltpu.sync_copy(rows0.at[pl.ds(0, rem)],
                      acc.at[pl.ds(base + nfull * _CH, rem)])
    if with_deg:
      def zero_o(i, carry):
        obuf[i, pl.ds(0, 16)] = z
        return carry

      lax.fori_loop(0, _CH, zero_o, 0)
      for k in range(nfull):
        pltpu.sync_copy(obuf, acc16.at[pl.ds(base + k * _CH, _CH)])
      if rem:
        pltpu.sync_copy(obuf.at[pl.ds(0, rem)],
                        acc16.at[pl.ds(base + nfull * _CH, rem)])
      one = jnp.ones((16,), jnp.float32)

      def ones_o(i, carry):
        obuf[i, pl.ds(0, 16)] = one
        return carry

      lax.fori_loop(0, _CH, ones_o, 0)
    plsc.subcore_barrier()

    def gstart(j, b):
      pltpu.async_copy(x_hbm.at[idxsrc.at[j]], rbuf[b], sems[b])

    def gwait(j, b):
      pltpu.make_async_copy(x_hbm.at[idxsrc.at[j]], rbuf[b], sems[b]).wait()

    def scat(j, b):
      pltpu.sync_copy(rbuf[b], acc.at[idxdst.at[j]], add=True)

    ih = _CPW // 2
    for half in range(2):
      pltpu.sync_copy(edge_hbm.at[0, pl.ds(w * _CPW + half * ih, ih)], idxsrc)
      pltpu.sync_copy(edge_hbm.at[1, pl.ds(w * _CPW + half * ih, ih)], idxdst)
      gstart(0, 0)
      gstart(1, 1)

      def dfire(j):
        if with_deg:
          pltpu.async_copy(obuf, acc16.at[idxdst.at[j]], sem2, add=True)

      def body(t, carry):
        j = t * 2
        gwait(j, 0)
        scat(j, 0)
        dfire(j)
        gstart(j + 2, 0)
        gwait(j + 1, 1)
        scat(j + 1, 1)
        dfire(j + 1)
        gstart(j + 3, 1)
        return carry

      lax.fori_loop(0, ih // 2 - 1, body, 0)
      gwait(ih - 2, 0)
      scat(ih - 2, 0)
      dfire(ih - 2)
      gwait(ih - 1, 1)
      scat(ih - 1, 1)
      dfire(ih - 1)

      if with_deg:
        def ddrain(j, carry):
          pltpu.make_async_copy(obuf, acc16.at[idxdst.at[0]], sem2).wait()
          return carry

        lax.fori_loop(0, ih, ddrain, 0)
    plsc.subcore_barrier()

    for k in range(nfull):
      pltpu.sync_copy(acc.at[pl.ds(base + k * _CH, _CH)], rows0)
      pltpu.sync_copy(rows0, out_hbm.at[c, pl.ds(base + k * _CH, _CH)])
    if rem:
      off = base + nfull * _CH
      pltpu.sync_copy(acc.at[pl.ds(off, rem)], rows0.at[pl.ds(0, rem)])
      pltpu.sync_copy(rows0.at[pl.ds(0, rem)], out_hbm.at[c, pl.ds(off, rem)])
    if with_deg:
      for k in range(nfull):
        pltpu.sync_copy(acc16.at[pl.ds(base + k * _CH, _CH)], obuf)
        pltpu.sync_copy(obuf, out16_hbm.at[c, pl.ds(base + k * _CH, _CH)])
      if rem:
        off = base + nfull * _CH
        pltpu.sync_copy(acc16.at[pl.ds(off, rem)], obuf.at[pl.ds(0, rem)])
        pltpu.sync_copy(obuf.at[pl.ds(0, rem)],
                        out16_hbm.at[c, pl.ds(off, rem)])

  return seg


_seg1 = _make_segsum(True)
_seg2 = _make_segsum(False)


def _mm_a(h, W1, b1):
  def body(h_ref, w_ref, b_ref, o_ref):
    o_ref[...] = jnp.dot(h_ref[...], w_ref[...],
                         preferred_element_type=jnp.float32) + b_ref[...]

  return pl.pallas_call(
      body,
      grid=(_N // _BN,),
      in_specs=[
          pl.BlockSpec((_BN, _D), lambda i: (i, 0)),
          pl.BlockSpec((_D, _D), lambda i: (0, 0)),
          pl.BlockSpec((1, _D), lambda i: (0, 0)),
      ],
      out_specs=pl.BlockSpec((_BN, _D), lambda i: (i, 0)),
      out_shape=jax.ShapeDtypeStruct((_N, _D), jnp.float32),
  )(h, W1, b1)


def _mm_b(acc1, deg16, W2, b2):
  def body(a_ref, d_ref, w_ref, b_ref, x2_ref, deg_ref):
    a = a_ref[...]
    d = d_ref[...]
    deg = jnp.maximum(d[0, :, 0:1] + d[1, :, 0:1], 1.0)
    h1 = jnp.maximum((a[0] + a[1]) / deg, 0.0)
    x2_ref[...] = jnp.dot(h1, w_ref[...],
                          preferred_element_type=jnp.float32) + b_ref[...]
    deg_ref[...] = jnp.broadcast_to(deg, (_BN, _D))

  return pl.pallas_call(
      body,
      grid=(_N // _BN,),
      in_specs=[
          pl.BlockSpec((_NC, _BN, _D), lambda i: (0, i, 0)),
          pl.BlockSpec((_NC, _BN, 16), lambda i: (0, i, 0)),
          pl.BlockSpec((_D, _D), lambda i: (0, 0)),
          pl.BlockSpec((1, _D), lambda i: (0, 0)),
      ],
      out_specs=[
          pl.BlockSpec((_BN, _D), lambda i: (i, 0)),
          pl.BlockSpec((_BN, _D), lambda i: (i, 0)),
      ],
      out_shape=[
          jax.ShapeDtypeStruct((_N, _D), jnp.float32),
          jax.ShapeDtypeStruct((_N, _D), jnp.float32),
      ],
  )(acc1, deg16, W2, b2)


def _mm_c(acc2, degb):
  def body(a_ref, deg_ref, o_ref):
    a = a_ref[...]
    s2 = a[0] + a[1]
    h2 = jnp.maximum(s2 / deg_ref[...], 0.0)
    nrm = jnp.sqrt(jnp.sum(h2 * h2, axis=1, keepdims=True))
    o_ref[...] = h2 / jnp.maximum(nrm, 1e-12)

  return pl.pallas_call(
      body,
      grid=(_N // _BN,),
      in_specs=[
          pl.BlockSpec((_NC, _BN, _D), lambda i: (0, i, 0)),
          pl.BlockSpec((_BN, _D), lambda i: (i, 0)),
      ],
      out_specs=pl.BlockSpec((_BN, _D), lambda i: (i, 0)),
      out_shape=jax.ShapeDtypeStruct((_N, _D), jnp.float32),
  )(acc2, degb)


def kernel(h, edge_index, W1, b1, W2, b2):
  edge3 = edge_index.reshape(2, _E // _CH, _CH)
  x1 = _mm_a(h, W1, b1.reshape(1, _D))
  acc1, deg16 = _seg1(x1, edge3)
  x2, degb = _mm_b(acc1, deg16, W2, b2.reshape(1, _D))
  acc2, = _seg2(x2, edge3)
  return _mm_c(acc2, degb)
